# baseline (speedup 1.0000x reference)
import functools

import jax
import jax.numpy as jnp
from jax import lax
from jax.experimental import pallas as pl
from jax.experimental.pallas import tpu as pltpu
from jax.experimental.pallas import tpu_sc as plsc

B = 4096
H = 200
D = 32

_info = plsc.get_sparse_core_info()
NC = _info.num_cores
NS = _info.num_subcores
NW = NC * NS
RW = B // NW
NBUF = 4
L = 16

_mesh = plsc.VectorSubcoreMesh(core_axis_name="c", subcore_axis_name="s")


@functools.partial(
    pl.kernel,
    mesh=_mesh,
    out_type=jax.ShapeDtypeStruct((H, D, B), jnp.float32),
    scratch_types=[
        pltpu.VMEM((RW, H), jnp.int32),
        pltpu.VMEM((H, RW), jnp.int32),
        pltpu.VMEM((NBUF, RW, D), jnp.float32),
        pltpu.VMEM((NBUF, D, RW), jnp.float32),
        [pltpu.SemaphoreType.DMA] * NBUF,
        [pltpu.SemaphoreType.DMA] * NBUF,
    ],
    compiler_params=pltpu.CompilerParams(use_tc_tiling_on_sc=False, needs_layout_passes=False),
)
def _gather(idx_hbm, table_hbm, out_hbm, idx_v, idxt_v, rows_v, tr_v, gsem, ssem):
    wid = lax.axis_index("s") * NC + lax.axis_index("c")
    base = wid * RW

    pltpu.sync_copy(idx_hbm.at[pl.ds(base, RW)], idx_v)

    lane = lax.broadcasted_iota(jnp.int32, (L,), 0)

    @pl.loop(0, H)
    def _th(h):
        for j in range(RW // L):
            v = plsc.load_gather(idx_v, [j * L + lane, jnp.full((L,), 0, jnp.int32) + h])
            idxt_v[h, pl.ds(j * L, L)] = v

    def g_issue(h, b):
        pltpu.async_copy(table_hbm.at[idxt_v.at[h]], rows_v.at[b], gsem[b])

    def g_wait(b):
        pltpu.make_async_copy(table_hbm.at[pl.ds(0, RW)], rows_v.at[b], gsem[b]).wait()

    def s_issue(h, b):
        pltpu.async_copy(tr_v.at[b], out_hbm.at[h, :, pl.ds(base, RW)], ssem[b])

    def s_wait(b):
        pltpu.make_async_copy(tr_v.at[b], out_hbm.at[0, :, pl.ds(0, RW)], ssem[b]).wait()

    def transpose_block(b):
        for d in range(D):
            dvec = jnp.full((L,), d, jnp.int32)
            for j in range(RW // L):
                v = plsc.load_gather(rows_v.at[b], [j * L + lane, dvec])
                tr_v[b, d, pl.ds(j * L, L)] = v

    for b in range(NBUF):
        g_issue(b, b)

    @pl.loop(0, H // NBUF - 1)
    def _round(k):
        h0 = k * NBUF
        for b in range(NBUF):
            g_wait(b)
            transpose_block(b)
            s_wait_if = b
            s_issue(h0 + b, b)
            s_wait(b)
            g_issue(h0 + NBUF + b, b)

    h0 = H - NBUF
    for b in range(NBUF):
        g_wait(b)
        transpose_block(b)
        s_issue(h0 + b, b)
    for b in range(NBUF):
        s_wait(b)


def kernel(ego_feature, token_table):
    out = _gather(ego_feature, token_table)
    return jnp.transpose(out, (2, 0, 1))


# speedup vs baseline: 1.3523x; 1.3523x over previous
import functools

import jax
import jax.numpy as jnp
from jax import lax
from jax.experimental import pallas as pl
from jax.experimental.pallas import tpu as pltpu
from jax.experimental.pallas import tpu_sc as plsc

B = 4096
H = 200
D = 32

_info = plsc.get_sparse_core_info()
NC = _info.num_cores
NS = _info.num_subcores
NW = NC * NS
ROWS_W = B // NW
NBUF = 8

_mesh = plsc.VectorSubcoreMesh(core_axis_name="c", subcore_axis_name="s")


@functools.partial(
    pl.kernel,
    mesh=_mesh,
    out_type=jax.ShapeDtypeStruct((B, H, D), jnp.float32),
    scratch_types=[
        pltpu.VMEM((ROWS_W, H), jnp.int32),
        pltpu.VMEM((NBUF, H, D), jnp.float32),
        [pltpu.SemaphoreType.DMA] * NBUF,
        [pltpu.SemaphoreType.DMA] * NBUF,
    ],
    compiler_params=pltpu.CompilerParams(use_tc_tiling_on_sc=False),
)
def _gather(idx_hbm, table_hbm, out_hbm, idx_v, rows_v, gsem, ssem):
    wid = lax.axis_index("s") * NC + lax.axis_index("c")
    base = wid * ROWS_W

    pltpu.sync_copy(idx_hbm.at[pl.ds(base, ROWS_W)], idx_v)

    def g_issue(r, b):
        pltpu.async_copy(table_hbm.at[idx_v.at[r]], rows_v.at[b], gsem[b])

    def g_wait(b):
        pltpu.make_async_copy(out_hbm.at[0], rows_v.at[b], gsem[b]).wait()

    def s_issue(r, b):
        pltpu.async_copy(rows_v.at[b], out_hbm.at[base + r], ssem[b])

    def s_wait(b):
        pltpu.make_async_copy(rows_v.at[b], out_hbm.at[0], ssem[b]).wait()

    for b in range(NBUF):
        g_issue(b, b)

    @pl.loop(0, ROWS_W // NBUF - 1)
    def _round(k):
        r0 = k * NBUF
        for b in range(NBUF):
            g_wait(b)
            s_issue(r0 + b, b)
            s_wait(b)
            g_issue(r0 + NBUF + b, b)

    r0 = ROWS_W - NBUF
    for b in range(NBUF):
        g_wait(b)
        s_issue(r0 + b, b)
    for b in range(NBUF):
        s_wait(b)


def kernel(ego_feature, token_table):
    return _gather(ego_feature, token_table)
